# baseline (speedup 1.0000x reference)
import functools

import jax
import jax.numpy as jnp
from jax import lax
from jax.experimental import pallas as pl
from jax.experimental.pallas import tpu as pltpu
from jax.experimental.pallas import tpu_sc as plsc

_NC = 2
_NS = 16
_NW = _NC * _NS
_CHUNK = 80

_VT = 4096
_NBUF = 2
_LOG2E = 1.4426950408889634
_LN2 = 0.6931471805599453


def _gather_rows(emb, x_flat):
    n, d = x_flat.shape[0], emb.shape[1]
    per_w = n // _NW
    n_chunks = per_w // _CHUNK
    idx3 = x_flat.reshape(_NW, n_chunks, _CHUNK)
    mesh = plsc.VectorSubcoreMesh(core_axis_name="c", subcore_axis_name="s")

    @functools.partial(
        pl.kernel,
        mesh=mesh,
        out_type=jax.ShapeDtypeStruct((n, d), jnp.float32),
        compiler_params=pltpu.CompilerParams(use_tc_tiling_on_sc=False),
        scratch_types=[
            pltpu.VMEM((n_chunks, _CHUNK), jnp.int32),
            pltpu.VMEM((per_w, d), jnp.float32),
            pltpu.SemaphoreType.DMA,
        ],
    )
    def sc_gather(table_hbm, idx_hbm, out_hbm, idx_v, rows_v, sem):
        wid = lax.axis_index("s") * _NC + lax.axis_index("c")
        pltpu.sync_copy(idx_hbm.at[wid], idx_v)

        def body(c, carry):
            pltpu.async_copy(
                table_hbm.at[idx_v.at[c]],
                rows_v.at[pl.ds(c * _CHUNK, _CHUNK), :],
                sem,
            ).wait()
            return carry

        lax.fori_loop(0, n_chunks, body, 0)
        pltpu.sync_copy(rows_v, out_hbm.at[pl.ds(wid * per_w, per_w)])

    return sc_gather(emb, idx3)


def _fc1_body(e_ref, w1_ref, b1_ref, h_ref):
    acc = lax.dot_general(
        e_ref[...], w1_ref[...], (((1,), (1,)), ((), ())),
        preferred_element_type=jnp.float32,
    )
    h_ref[...] = jnp.maximum(acc + b1_ref[...], 0.0)


def _lse_body(h_ref, w2_ref, b2s_ref, m_ref, s_ref, h2_ref, w2b_ref):
    j = pl.program_id(0)

    @pl.when(j == 0)
    def _init():
        h2_ref[...] = (h_ref[...] * _LOG2E).astype(jnp.bfloat16)
        m_ref[...] = jnp.full_like(m_ref, -jnp.inf)
        s_ref[...] = jnp.zeros_like(s_ref)

    w2b_ref[...] = w2_ref[...].astype(jnp.bfloat16)
    logits2 = lax.dot_general(
        h2_ref[...], w2b_ref[...], (((1,), (1,)), ((), ())),
        preferred_element_type=jnp.float32,
    ) + b2s_ref[...]
    m_old = m_ref[...]
    m_new = jnp.maximum(m_old, jnp.max(logits2, axis=1, keepdims=True))
    s_ref[...] = s_ref[...] * jnp.exp2(m_old - m_new) + jnp.sum(
        jnp.exp2(logits2 - m_new), axis=1, keepdims=True)
    m_ref[...] = m_new


def _lse_tail_body(h_ref, w2t_ref, b2ts_ref, m_ref, s_ref, l_ref):
    logits2 = lax.dot_general(
        h_ref[...] * _LOG2E, w2t_ref[...], (((1,), (1,)), ((), ())),
        preferred_element_type=jnp.float32,
    ) + b2ts_ref[...]
    m_old = m_ref[...]
    m_new = jnp.maximum(m_old, jnp.max(logits2, axis=1, keepdims=True))
    s_new = s_ref[...] * jnp.exp2(m_old - m_new) + jnp.sum(
        jnp.exp2(logits2 - m_new), axis=1, keepdims=True)
    l_ref[...] = m_new * _LN2 + jnp.log(s_new)


def _make_out_body(n_full):
    def _out_body(h_ref, w2_ref, b2_ref, l_ref, o_hbm, buf, sem):
        j = pl.program_id(0)
        slot = lax.rem(j, _NBUF)

        @pl.when(j >= _NBUF)
        def _wait_prev():
            pltpu.make_async_copy(
                buf.at[slot],
                o_hbm.at[:, pl.ds((j - _NBUF) * _VT, _VT)],
                sem.at[slot],
            ).wait()

        logits = lax.dot_general(
            h_ref[...], w2_ref[...], (((1,), (1,)), ((), ())),
            preferred_element_type=jnp.float32,
        ) + b2_ref[...]
        buf[slot] = logits - l_ref[...]
        pltpu.make_async_copy(
            buf.at[slot], o_hbm.at[:, pl.ds(j * _VT, _VT)], sem.at[slot]
        ).start()

        @pl.when(j == n_full - 1)
        def _drain():
            for k in range(min(_NBUF, n_full)):
                jj = n_full - 1 - k
                s = jj % _NBUF
                pltpu.make_async_copy(
                    buf.at[s], o_hbm.at[:, pl.ds(jj * _VT, _VT)], sem.at[s]
                ).wait()

    return _out_body


def _tail_body(h_ref, w2_ref, b2_ref, l_ref, o_full_ref, o_ref):
    del o_full_ref
    logits = lax.dot_general(
        h_ref[...], w2_ref[...], (((1,), (1,)), ((), ())),
        preferred_element_type=jnp.float32,
    ) + b2_ref[...]
    o_ref[...] = logits - l_ref[...]


def kernel(x, emb, W1, b1, W2, b2):
    batch, hist = x.shape
    v, d = emb.shape
    hid = W1.shape[0]

    e_flat = _gather_rows(emb, x.reshape(-1))
    embeds = e_flat.reshape(batch, hist * d)

    h = pl.pallas_call(
        _fc1_body,
        out_shape=jax.ShapeDtypeStruct((batch, hid), jnp.float32),
    )(embeds, W1, b1.reshape(1, hid))

    n_tiles = pl.cdiv(v, _VT)
    n_full = v // _VT
    tail_w = v - n_full * _VT
    vpad = n_tiles * _VT - v
    b2_row = jnp.pad(b2.reshape(1, v), ((0, 0), (0, vpad)),
                     constant_values=-jnp.inf)

    m, s = pl.pallas_call(
        _lse_body,
        grid=(n_full,),
        in_specs=[
            pl.BlockSpec((batch, hid), lambda j: (0, 0)),
            pl.BlockSpec((_VT, hid), lambda j: (j, 0)),
            pl.BlockSpec((1, _VT), lambda j: (0, j)),
        ],
        out_specs=[
            pl.BlockSpec((batch, 1), lambda j: (0, 0)),
            pl.BlockSpec((batch, 1), lambda j: (0, 0)),
        ],
        out_shape=[
            jax.ShapeDtypeStruct((batch, 1), jnp.float32),
            jax.ShapeDtypeStruct((batch, 1), jnp.float32),
        ],
        scratch_shapes=[
            pltpu.VMEM((batch, hid), jnp.bfloat16),
            pltpu.VMEM((_VT, hid), jnp.bfloat16),
        ],
    )(h, W2, (b2.reshape(1, v) * _LOG2E)[:, :n_full * _VT])

    w2_tail = lax.slice(W2, (n_full * _VT, 0), (v, hid))
    b2_tail = lax.slice(b2.reshape(1, v), (0, n_full * _VT), (1, v)) * _LOG2E
    lse = pl.pallas_call(
        _lse_tail_body,
        out_shape=jax.ShapeDtypeStruct((batch, 1), jnp.float32),
    )(h, w2_tail, b2_tail, m, s)
    out_main = pl.pallas_call(
        _make_out_body(n_full),
        grid=(n_full,),
        in_specs=[
            pl.BlockSpec((batch, hid), lambda j: (0, 0)),
            pl.BlockSpec((_VT, hid), lambda j: (j, 0)),
            pl.BlockSpec((1, _VT), lambda j: (0, j)),
            pl.BlockSpec((batch, 1), lambda j: (0, 0)),
        ],
        out_specs=pl.BlockSpec(memory_space=pltpu.MemorySpace.HBM),
        out_shape=jax.ShapeDtypeStruct((batch, v), jnp.float32),
        scratch_shapes=[
            pltpu.VMEM((_NBUF, batch, _VT), jnp.float32),
            pltpu.SemaphoreType.DMA((_NBUF,)),
        ],
    )(h, W2, b2_row, lse)

    jt = n_tiles - 1
    out = pl.pallas_call(
        _tail_body,
        grid=(1,),
        in_specs=[
            pl.BlockSpec((batch, hid), lambda j: (0, 0)),
            pl.BlockSpec((_VT, hid), lambda j: (jt, 0)),
            pl.BlockSpec((1, _VT), lambda j: (0, jt)),
            pl.BlockSpec((batch, 1), lambda j: (0, 0)),
            pl.BlockSpec(memory_space=pltpu.MemorySpace.HBM),
        ],
        out_specs=pl.BlockSpec((batch, _VT), lambda j: (0, jt)),
        out_shape=jax.ShapeDtypeStruct((batch, v), jnp.float32),
        input_output_aliases={4: 0},
    )(h, W2, b2_row, lse, out_main)

    return out


# speedup vs baseline: 1.0073x; 1.0073x over previous
import functools

import jax
import jax.numpy as jnp
from jax import lax
from jax.experimental import pallas as pl
from jax.experimental.pallas import tpu as pltpu
from jax.experimental.pallas import tpu_sc as plsc

_NC = 2
_NS = 16
_NW = _NC * _NS
_CHUNK = 80

_VT = 4096
_NBUF = 2
_LOG2E = 1.4426950408889634
_LN2 = 0.6931471805599453


def _gather_rows(emb, x_flat):
    n, d = x_flat.shape[0], emb.shape[1]
    per_w = n // _NW
    n_chunks = per_w // _CHUNK
    idx3 = x_flat.reshape(_NW, n_chunks, _CHUNK)
    mesh = plsc.VectorSubcoreMesh(core_axis_name="c", subcore_axis_name="s")

    @functools.partial(
        pl.kernel,
        mesh=mesh,
        out_type=jax.ShapeDtypeStruct((n, d), jnp.float32),
        compiler_params=pltpu.CompilerParams(use_tc_tiling_on_sc=False),
        scratch_types=[
            pltpu.VMEM((n_chunks, _CHUNK), jnp.int32),
            pltpu.VMEM((per_w, d), jnp.float32),
            pltpu.SemaphoreType.DMA,
        ],
    )
    def sc_gather(table_hbm, idx_hbm, out_hbm, idx_v, rows_v, sem):
        wid = lax.axis_index("s") * _NC + lax.axis_index("c")
        pltpu.sync_copy(idx_hbm.at[wid], idx_v)

        def body(c, carry):
            pltpu.async_copy(
                table_hbm.at[idx_v.at[c]],
                rows_v.at[pl.ds(c * _CHUNK, _CHUNK), :],
                sem,
            ).wait()
            return carry

        lax.fori_loop(0, n_chunks, body, 0)
        pltpu.sync_copy(rows_v, out_hbm.at[pl.ds(wid * per_w, per_w)])

    return sc_gather(emb, idx3)


def _fc1_body(e_ref, w1_ref, b1_ref, h_ref):
    acc = lax.dot_general(
        e_ref[...], w1_ref[...], (((1,), (1,)), ((), ())),
        preferred_element_type=jnp.float32,
    )
    h_ref[...] = jnp.maximum(acc + b1_ref[...], 0.0)


def _lse_body(h_ref, w2_ref, b2s_ref, m_ref, s_ref, h2_ref, w2b_ref):
    j = pl.program_id(0)

    @pl.when(j == 0)
    def _init():
        h2_ref[...] = (h_ref[...] * _LOG2E).astype(jnp.bfloat16)
        m_ref[...] = jnp.full_like(m_ref, -jnp.inf)
        s_ref[...] = jnp.zeros_like(s_ref)

    w2b_ref[...] = w2_ref[...].astype(jnp.bfloat16)
    logits2 = lax.dot_general(
        h2_ref[...], w2b_ref[...], (((1,), (1,)), ((), ())),
        preferred_element_type=jnp.float32,
    ) + b2s_ref[...]
    m_old = m_ref[...]
    m_new = jnp.maximum(m_old, jnp.max(logits2, axis=1, keepdims=True))
    s_ref[...] = s_ref[...] * jnp.exp2(m_old - m_new) + jnp.sum(
        jnp.exp2(logits2 - m_new), axis=1, keepdims=True)
    m_ref[...] = m_new


def _lse_tail_body(h_ref, w2t_ref, b2ts_ref, m_ref, s_ref, l_ref):
    logits2 = lax.dot_general(
        h_ref[...] * _LOG2E, w2t_ref[...], (((1,), (1,)), ((), ())),
        preferred_element_type=jnp.float32,
    ) + b2ts_ref[...]
    m_old = m_ref[...]
    m_new = jnp.maximum(m_old, jnp.max(logits2, axis=1, keepdims=True))
    s_new = s_ref[...] * jnp.exp2(m_old - m_new) + jnp.sum(
        jnp.exp2(logits2 - m_new), axis=1, keepdims=True)
    l_ref[...] = m_new * _LN2 + jnp.log(s_new)


def _make_out_body(n_full):
    def _out_body(h_ref, w2_ref, b2_ref, l_ref, o_hbm, buf, sem):
        j = pl.program_id(0)
        slot = lax.rem(j, _NBUF)

        @pl.when(j >= _NBUF)
        def _wait_prev():
            pltpu.make_async_copy(
                buf.at[slot],
                o_hbm.at[:, pl.ds((j - _NBUF) * _VT, _VT)],
                sem.at[slot],
            ).wait()

        logits = lax.dot_general(
            h_ref[...], w2_ref[...], (((1,), (1,)), ((), ())),
            preferred_element_type=jnp.float32,
        ) + b2_ref[...]
        buf[slot] = logits - l_ref[...]
        pltpu.make_async_copy(
            buf.at[slot], o_hbm.at[:, pl.ds(j * _VT, _VT)], sem.at[slot]
        ).start()

        @pl.when(j == n_full - 1)
        def _drain():
            for k in range(min(_NBUF, n_full)):
                jj = n_full - 1 - k
                s = jj % _NBUF
                pltpu.make_async_copy(
                    buf.at[s], o_hbm.at[:, pl.ds(jj * _VT, _VT)], sem.at[s]
                ).wait()

    return _out_body


def _tail_body(h_ref, w2_ref, b2_ref, l_ref, o_full_ref, o_ref):
    del o_full_ref
    logits = lax.dot_general(
        h_ref[...], w2_ref[...], (((1,), (1,)), ((), ())),
        preferred_element_type=jnp.float32,
    ) + b2_ref[...]
    o_ref[...] = logits - l_ref[...]


def kernel(x, emb, W1, b1, W2, b2):
    batch, hist = x.shape
    v, d = emb.shape
    hid = W1.shape[0]

    e_flat = _gather_rows(emb, x.reshape(-1))
    embeds = e_flat.reshape(batch, hist * d)

    h = pl.pallas_call(
        _fc1_body,
        out_shape=jax.ShapeDtypeStruct((batch, hid), jnp.float32),
    )(embeds, W1, b1.reshape(1, hid))

    n_tiles = pl.cdiv(v, _VT)
    n_full = v // _VT
    tail_w = v - n_full * _VT
    vpad = n_tiles * _VT - v
    b2_row = jnp.pad(b2.reshape(1, v), ((0, 0), (0, vpad)),
                     constant_values=-jnp.inf)

    vt1 = 2 * _VT
    m, s = pl.pallas_call(
        _lse_body,
        grid=(n_full // 2,),
        in_specs=[
            pl.BlockSpec((batch, hid), lambda j: (0, 0)),
            pl.BlockSpec((vt1, hid), lambda j: (j, 0)),
            pl.BlockSpec((1, vt1), lambda j: (0, j)),
        ],
        out_specs=[
            pl.BlockSpec((batch, 1), lambda j: (0, 0)),
            pl.BlockSpec((batch, 1), lambda j: (0, 0)),
        ],
        out_shape=[
            jax.ShapeDtypeStruct((batch, 1), jnp.float32),
            jax.ShapeDtypeStruct((batch, 1), jnp.float32),
        ],
        scratch_shapes=[
            pltpu.VMEM((batch, hid), jnp.bfloat16),
            pltpu.VMEM((2 * _VT, hid), jnp.bfloat16),
        ],
    )(h, W2, (b2.reshape(1, v) * _LOG2E)[:, :n_full * _VT])

    w2_tail = lax.slice(W2, (n_full * _VT, 0), (v, hid))
    b2_tail = lax.slice(b2.reshape(1, v), (0, n_full * _VT), (1, v)) * _LOG2E
    lse = pl.pallas_call(
        _lse_tail_body,
        out_shape=jax.ShapeDtypeStruct((batch, 1), jnp.float32),
    )(h, w2_tail, b2_tail, m, s)
    out_main = pl.pallas_call(
        _make_out_body(n_full),
        grid=(n_full,),
        in_specs=[
            pl.BlockSpec((batch, hid), lambda j: (0, 0)),
            pl.BlockSpec((_VT, hid), lambda j: (j, 0)),
            pl.BlockSpec((1, _VT), lambda j: (0, j)),
            pl.BlockSpec((batch, 1), lambda j: (0, 0)),
        ],
        out_specs=pl.BlockSpec(memory_space=pltpu.MemorySpace.HBM),
        out_shape=jax.ShapeDtypeStruct((batch, v), jnp.float32),
        scratch_shapes=[
            pltpu.VMEM((_NBUF, batch, _VT), jnp.float32),
            pltpu.SemaphoreType.DMA((_NBUF,)),
        ],
    )(h, W2, b2_row, lse)

    jt = n_tiles - 1
    out = pl.pallas_call(
        _tail_body,
        grid=(1,),
        in_specs=[
            pl.BlockSpec((batch, hid), lambda j: (0, 0)),
            pl.BlockSpec((_VT, hid), lambda j: (jt, 0)),
            pl.BlockSpec((1, _VT), lambda j: (0, jt)),
            pl.BlockSpec((batch, 1), lambda j: (0, 0)),
            pl.BlockSpec(memory_space=pltpu.MemorySpace.HBM),
        ],
        out_specs=pl.BlockSpec((batch, _VT), lambda j: (0, jt)),
        out_shape=jax.ShapeDtypeStruct((batch, v), jnp.float32),
        input_output_aliases={4: 0},
    )(h, W2, b2_row, lse, out_main)

    return out
